# baseline (speedup 1.0000x reference)
import functools

import jax
import jax.numpy as jnp
from jax import lax
from jax.experimental import pallas as pl
from jax.experimental.pallas import tpu as pltpu
from jax.experimental.pallas import tpu_sc as plsc

NROWS = 16384
NCOLS = 720
NOUT = 256
NLANES = 16
NC = 2
NS = 16
NW = NC * NS
RPW = NROWS // NW
RB = 32
NB = RPW // RB
NG = NOUT // NLANES
NIN = 4
NOUTB = 2

_mesh = plsc.VectorSubcoreMesh(core_axis_name="c", subcore_axis_name="s")


@functools.partial(
    pl.kernel,
    out_type=jax.ShapeDtypeStruct((NROWS * NOUT,), jnp.float32),
    mesh=_mesh,
    compiler_params=pltpu.CompilerParams(needs_layout_passes=False),
    scratch_types=[
        pltpu.VMEM((NOUT,), jnp.int32),
        pltpu.VMEM((NIN, RB * NCOLS), jnp.float32),
        pltpu.VMEM((NOUTB, RB * NOUT), jnp.float32),
        pltpu.SemaphoreType.DMA,
        pltpu.SemaphoreType.DMA,
        pltpu.SemaphoreType.DMA,
        pltpu.SemaphoreType.DMA,
        pltpu.SemaphoreType.DMA,
        pltpu.SemaphoreType.DMA,
    ],
)
def _gather_k(x_hbm, idx_hbm, out_hbm, idx_v, in_v, out_v,
              si0, si1, si2, si3, so0, so1):
    wid = lax.axis_index("s") * NC + lax.axis_index("c")
    row0 = wid * RPW

    pltpu.sync_copy(idx_hbm, idx_v)
    idxr = [idx_v[pl.ds(NLANES * g, NLANES)] for g in range(NG)]

    sin = (si0, si1, si2, si3)
    sout = (so0, so1)

    def in_src(blk):
        return x_hbm.at[pl.ds((row0 + blk * RB) * NCOLS, RB * NCOLS)]

    def out_dst(blk):
        return out_hbm.at[pl.ds((row0 + blk * RB) * NOUT, RB * NOUT)]

    for blk in range(NIN - 1):
        pltpu.async_copy(in_src(blk), in_v.at[blk], sin[blk])

    for blk in range(NB):
        b = blk % NIN
        ob = blk % NOUTB
        if blk + NIN - 1 < NB:
            nb = (blk + NIN - 1) % NIN
            pltpu.async_copy(in_src(blk + NIN - 1), in_v.at[nb], sin[nb])
        pltpu.make_async_copy(in_src(blk), in_v.at[b], sin[b]).wait()
        if blk >= NOUTB:
            pltpu.make_async_copy(out_v.at[ob], out_dst(blk - NOUTB),
                                  sout[ob]).wait()

        @plsc.parallel_loop(0, RB, 1, unroll=2)
        def row_body(r, b=b, ob=ob):
            cbase = r * NCOLS
            obase = r * NOUT
            bvec = jnp.full((NLANES,), b, dtype=jnp.int32)
            for g in range(NG):
                val = plsc.load_gather(in_v, [bvec, idxr[g] + cbase])
                out_v[ob, pl.ds(obase + NLANES * g, NLANES)] = val

        pltpu.async_copy(out_v.at[ob], out_dst(blk), sout[ob])

    for blk in range(NB - NOUTB, NB):
        ob = blk % NOUTB
        pltpu.make_async_copy(out_v.at[ob], out_dst(blk), sout[ob]).wait()


def kernel(input, inds_idx):
    x = input.reshape(NROWS * NCOLS)
    out = _gather_k(x, inds_idx)
    return out.reshape(NROWS, NOUT)


# speedup vs baseline: 51.5215x; 51.5215x over previous
import functools

import jax
import jax.numpy as jnp
from jax import lax
from jax.experimental import pallas as pl
from jax.experimental.pallas import tpu as pltpu
from jax.experimental.pallas import tpu_sc as plsc

NROWS = 16384
NCOLS = 720
NOUT = 256
NLANES = 16
NC = 2
NS = 16
NW = NC * NS
RPW = NROWS // NW
RB = 32
NB = RPW // RB
NG = NOUT // NLANES
NIN = 4
NOUTB = 2

_mesh = plsc.VectorSubcoreMesh(core_axis_name="c", subcore_axis_name="s")


@functools.partial(
    pl.kernel,
    out_type=jax.ShapeDtypeStruct((NROWS, NOUT), jnp.float32),
    mesh=_mesh,
    compiler_params=pltpu.CompilerParams(needs_layout_passes=False),
    scratch_types=[
        pltpu.VMEM((NOUT,), jnp.int32),
        pltpu.VMEM((RB, NCOLS), jnp.float32),
        pltpu.VMEM((RB, NCOLS), jnp.float32),
        pltpu.VMEM((RB, NCOLS), jnp.float32),
        pltpu.VMEM((RB, NCOLS), jnp.float32),
        pltpu.VMEM((RB, NOUT), jnp.float32),
        pltpu.VMEM((RB, NOUT), jnp.float32),
        pltpu.SemaphoreType.DMA,
        pltpu.SemaphoreType.DMA,
        pltpu.SemaphoreType.DMA,
        pltpu.SemaphoreType.DMA,
        pltpu.SemaphoreType.DMA,
        pltpu.SemaphoreType.DMA,
    ],
)
def _gather_k(x_hbm, idx_hbm, out_hbm, idx_v, in0, in1, in2, in3, o0, o1,
              si0, si1, si2, si3, so0, so1):
    wid = lax.axis_index("s") * NC + lax.axis_index("c")
    row0 = wid * RPW

    pltpu.sync_copy(idx_hbm, idx_v)
    idxr = [idx_v[pl.ds(NLANES * g, NLANES)] for g in range(NG)]

    ins = (in0, in1, in2, in3)
    outs = (o0, o1)
    sin = (si0, si1, si2, si3)
    sout = (so0, so1)

    def in_src(blk):
        return x_hbm.at[pl.ds(row0 + blk * RB, RB)]

    def out_dst(blk):
        return out_hbm.at[pl.ds(row0 + blk * RB, RB)]

    for blk in range(NIN - 1):
        pltpu.async_copy(in_src(blk), ins[blk], sin[blk])

    for blk in range(NB):
        b = blk % NIN
        ob = blk % NOUTB
        if blk + NIN - 1 < NB:
            nb = (blk + NIN - 1) % NIN
            pltpu.async_copy(in_src(blk + NIN - 1), ins[nb], sin[nb])
        pltpu.make_async_copy(in_src(blk), ins[b], sin[b]).wait()
        if blk >= NOUTB:
            pltpu.make_async_copy(outs[ob], out_dst(blk - NOUTB),
                                  sout[ob]).wait()

        in_v = ins[b]
        out_v = outs[ob]

        @plsc.parallel_loop(0, RB, 1, unroll=2)
        def row_body(r, in_v=in_v, out_v=out_v):
            rvec = jnp.full((NLANES,), r, dtype=jnp.int32)
            for g in range(NG):
                val = plsc.load_gather(in_v, [rvec, idxr[g]])
                out_v[r, pl.ds(NLANES * g, NLANES)] = val

        pltpu.async_copy(out_v, out_dst(blk), sout[ob])

    for blk in range(NB - NOUTB, NB):
        ob = blk % NOUTB
        pltpu.make_async_copy(outs[ob], out_dst(blk), sout[ob]).wait()


def kernel(input, inds_idx):
    x = input.reshape(NROWS, NCOLS)
    return _gather_k(x, inds_idx)
